# baseline (speedup 1.0000x reference)
import jax
import jax.numpy as jnp
from jax import lax
from jax.experimental import pallas as pl
from jax.experimental.pallas import tpu as pltpu
from jax.experimental.pallas import tpu_sc as plsc

VOCAB_SIZE = 1000
NUM_CORES = 2
NUM_SUBCORES = 16
NUM_WORKERS = NUM_CORES * NUM_SUBCORES
LANES = 16

TOTAL_ROWS = 4096 * 20
ROWS_PER_WORKER = TOTAL_ROWS // NUM_WORKERS
CHUNK_ROWS = 20
CHUNK_ELEMS = CHUNK_ROWS * VOCAB_SIZE
NUM_CHUNKS = ROWS_PER_WORKER // CHUNK_ROWS
NBUF = 4


def _sc_onehot(tok_hbm, out_hbm, *scratch):
    bufs = scratch[:NBUF]
    idx_v = scratch[NBUF]
    sems = scratch[NBUF + 1:]
    wid = lax.axis_index("s") * NUM_CORES + lax.axis_index("c")
    flat_base = wid * ROWS_PER_WORKER * VOCAB_SIZE
    tok_base = wid * ROWS_PER_WORKER
    pltpu.sync_copy(
        tok_hbm.at[pl.ds(tok_base, ROWS_PER_WORKER)],
        idx_v.at[pl.ds(0, ROWS_PER_WORKER)],
    )

    zeros16 = jnp.zeros((LANES,), jnp.float32)
    iota16 = lax.iota(jnp.int32, LANES)

    def zinit(i, _):
        for buf in bufs:
            buf[pl.ds(i * LANES, LANES)] = zeros16
        return 0

    lax.fori_loop(0, CHUNK_ELEMS // LANES, zinit, 0)

    def toks(g):
        return [
            idx_v[pl.ds(g * CHUNK_ROWS + k * LANES, LANES)]
            for k in range((CHUNK_ROWS + LANES - 1) // LANES)
        ]

    def marks(buf, g, value):
        tvs = toks(g)
        for r in range(CHUNK_ROWS):
            tok = tvs[r // LANES][r % LANES]
            p = r * VOCAB_SIZE + tok
            q = (p >> 4) << 4
            lane = p - q
            vec = buf[pl.ds(q, LANES)]
            vec = jnp.where(iota16 == lane, jnp.float32(value), vec)
            buf[pl.ds(q, LANES)] = vec

    def clear(buf, g):
        tvs = toks(g)
        for r in range(CHUNK_ROWS):
            tok = tvs[r // LANES][r % LANES]
            p = r * VOCAB_SIZE + tok
            q = (p >> 4) << 4
            buf[pl.ds(q, LANES)] = zeros16

    def dma(buf, sem, g):
        dst = out_hbm.at[pl.ds(flat_base + g * CHUNK_ELEMS, CHUNK_ELEMS)]
        return pltpu.make_async_copy(buf, dst, sem)

    for b in range(NBUF):
        marks(bufs[b], b, 1.0)
        dma(bufs[b], sems[b], b).start()

    def body(i, _):
        for b in range(NBUF):
            g = NBUF * i + b
            dma(bufs[b], sems[b], g - NBUF).wait()
            clear(bufs[b], g - NBUF)
            marks(bufs[b], g, 1.0)
            dma(bufs[b], sems[b], g).start()
        return 0

    lax.fori_loop(1, NUM_CHUNKS // NBUF, body, 0)

    for b in range(NBUF):
        dma(bufs[b], sems[b], NUM_CHUNKS - NBUF + b).wait()


_sc_kernel = pl.kernel(
    _sc_onehot,
    mesh=plsc.VectorSubcoreMesh(core_axis_name="c", subcore_axis_name="s"),
    out_type=jax.ShapeDtypeStruct((TOTAL_ROWS * VOCAB_SIZE,), jnp.float32),
    scratch_types=(
        [pltpu.VMEM((CHUNK_ELEMS,), jnp.float32) for _ in range(NBUF)]
        + [pltpu.VMEM((ROWS_PER_WORKER + LANES,), jnp.int32)]
        + [pltpu.SemaphoreType.DMA for _ in range(NBUF)]
    ),
)


_TC_BLOCK_ROWS = 512


def _tc_onehot_body(tok_ref, out_ref):
    tok = tok_ref[...]
    iot = lax.broadcasted_iota(jnp.int32, (1, VOCAB_SIZE), 1)
    out_ref[...] = (tok == iot).astype(jnp.float32)


_tc_kernel = pl.pallas_call(
    _tc_onehot_body,
    grid=(TOTAL_ROWS // _TC_BLOCK_ROWS,),
    in_specs=[
        pl.BlockSpec((_TC_BLOCK_ROWS, 1), lambda i: (i, 0)),
    ],
    out_specs=pl.BlockSpec((_TC_BLOCK_ROWS, VOCAB_SIZE), lambda i: (i, 0)),
    out_shape=jax.ShapeDtypeStruct((TOTAL_ROWS, VOCAB_SIZE), jnp.float32),
)


def kernel(tokens, table):
    del table
    flat_tok = tokens.reshape(-1)
    sc_flat = _sc_kernel(flat_tok)
    tc_part = _tc_kernel(flat_tok.reshape(-1, 1))
    half = TOTAL_ROWS // 2
    out = jnp.concatenate(
        [sc_flat.reshape(TOTAL_ROWS, VOCAB_SIZE)[:half], tc_part[half:]], axis=0
    )
    return out.reshape(tokens.shape[0], tokens.shape[1], VOCAB_SIZE)


# speedup vs baseline: 1.8552x; 1.8552x over previous
import jax
import jax.numpy as jnp
from jax import lax
from jax.experimental import pallas as pl
from jax.experimental.pallas import tpu as pltpu
from jax.experimental.pallas import tpu_sc as plsc

VOCAB_SIZE = 1000
NUM_CORES = 2
NUM_SUBCORES = 16
NUM_WORKERS = NUM_CORES * NUM_SUBCORES
LANES = 16

TOTAL_ROWS = 4096 * 20
ROWS_PER_WORKER = TOTAL_ROWS // NUM_WORKERS
ELEMS_PER_WORKER = ROWS_PER_WORKER * VOCAB_SIZE
ZBUF_ELEMS = 512_000
NUM_ZDMA = ELEMS_PER_WORKER // ZBUF_ELEMS
ZSTAGE_ELEMS = ZBUF_ELEMS // NUM_SUBCORES


def _sc_onehot(tok_hbm, out_hbm, zstage, idx_v, pos_v, ones_v, zbuf, *sems):
    zsems = sems[:NUM_ZDMA]
    ssem = sems[NUM_ZDMA]
    cid = lax.axis_index("c")
    sid = lax.axis_index("s")
    wid = sid * NUM_CORES + cid
    flat_base = wid * ELEMS_PER_WORKER
    tok_base = wid * ROWS_PER_WORKER
    pltpu.sync_copy(
        tok_hbm.at[pl.ds(tok_base, ROWS_PER_WORKER)],
        idx_v.at[pl.ds(0, ROWS_PER_WORKER)],
    )

    zeros16 = jnp.zeros((LANES,), jnp.float32)
    ones16 = jnp.ones((LANES,), jnp.float32)
    iota16 = lax.iota(jnp.int32, LANES)

    def zinit(i, _):
        zstage[pl.ds(i * LANES, LANES)] = zeros16
        return 0

    lax.fori_loop(0, ZSTAGE_ELEMS // LANES, zinit, 0)
    pltpu.sync_copy(zstage, zbuf.at[pl.ds(sid * ZSTAGE_ELEMS, ZSTAGE_ELEMS)])
    plsc.subcore_barrier()

    for d in range(NUM_ZDMA):
        pltpu.make_async_copy(
            zbuf,
            out_hbm.at[pl.ds(flat_base + d * ZBUF_ELEMS, ZBUF_ELEMS)],
            zsems[d],
        ).start()

    def posbody(k, _):
        rows = iota16 + (tok_base + k * LANES)
        toks = idx_v[pl.ds(k * LANES, LANES)]
        pos_v[pl.ds(k * LANES, LANES)] = rows * VOCAB_SIZE + toks
        ones_v[pl.ds(k * LANES, LANES)] = ones16
        return 0

    lax.fori_loop(0, ROWS_PER_WORKER // LANES, posbody, 0)

    for d in range(NUM_ZDMA):
        pltpu.make_async_copy(
            zbuf,
            out_hbm.at[pl.ds(flat_base + d * ZBUF_ELEMS, ZBUF_ELEMS)],
            zsems[d],
        ).wait()

    scat = pltpu.make_async_copy(ones_v, out_hbm.at[pos_v], ssem)
    scat.start()
    scat.wait()


_sc_kernel = pl.kernel(
    _sc_onehot,
    mesh=plsc.VectorSubcoreMesh(core_axis_name="c", subcore_axis_name="s"),
    out_type=jax.ShapeDtypeStruct((TOTAL_ROWS * VOCAB_SIZE,), jnp.float32),
    scratch_types=(
        [
            pltpu.VMEM((ZSTAGE_ELEMS,), jnp.float32),
            pltpu.VMEM((ROWS_PER_WORKER + LANES,), jnp.int32),
            pltpu.VMEM((ROWS_PER_WORKER,), jnp.int32),
            pltpu.VMEM((ROWS_PER_WORKER,), jnp.float32),
            pltpu.VMEM_SHARED((ZBUF_ELEMS,), jnp.float32),
        ]
        + [pltpu.SemaphoreType.DMA for _ in range(NUM_ZDMA + 1)]
    ),
)


def kernel(tokens, table):
    del table
    flat = _sc_kernel(tokens.reshape(-1))
    return flat.reshape(tokens.shape[0], tokens.shape[1], VOCAB_SIZE)


# speedup vs baseline: 2.0664x; 1.1139x over previous
import jax
import jax.numpy as jnp
from jax import lax
from jax.experimental import pallas as pl
from jax.experimental.pallas import tpu as pltpu
from jax.experimental.pallas import tpu_sc as plsc

VOCAB_SIZE = 1000
NUM_CORES = 2
NUM_SUBCORES = 16
NUM_WORKERS = NUM_CORES * NUM_SUBCORES
LANES = 16

TOTAL_ROWS = 4096 * 20
ROWS_PER_WORKER = TOTAL_ROWS // NUM_WORKERS
CHUNK_ROWS = 20
CHUNK_ELEMS = CHUNK_ROWS * VOCAB_SIZE
NUM_CHUNKS = ROWS_PER_WORKER // CHUNK_ROWS
NBUF = 4


def _sc_onehot(tok_hbm, out_hbm, *scratch):
    bufs = scratch[:NBUF]
    idx_v = scratch[NBUF]
    sems = scratch[NBUF + 1:]
    wid = lax.axis_index("s") * NUM_CORES + lax.axis_index("c")
    flat_base = wid * ROWS_PER_WORKER * VOCAB_SIZE
    tok_base = wid * ROWS_PER_WORKER
    pltpu.sync_copy(
        tok_hbm.at[pl.ds(tok_base, ROWS_PER_WORKER)],
        idx_v.at[pl.ds(0, ROWS_PER_WORKER)],
    )

    zeros16 = jnp.zeros((LANES,), jnp.float32)
    iota16 = lax.iota(jnp.int32, LANES)

    def zinit(i, _):
        for buf in bufs:
            buf[pl.ds(i * LANES, LANES)] = zeros16
        return 0

    lax.fori_loop(0, CHUNK_ELEMS // LANES, zinit, 0)

    def toks(g):
        return [
            idx_v[pl.ds(g * CHUNK_ROWS + k * LANES, LANES)]
            for k in range((CHUNK_ROWS + LANES - 1) // LANES)
        ]

    def marks(buf, g, value):
        tvs = toks(g)
        for r in range(CHUNK_ROWS):
            tok = tvs[r // LANES][r % LANES]
            p = r * VOCAB_SIZE + tok
            q = (p >> 4) << 4
            lane = p - q
            vec = buf[pl.ds(q, LANES)]
            vec = jnp.where(iota16 == lane, jnp.float32(value), vec)
            buf[pl.ds(q, LANES)] = vec

    def clear(buf, g):
        tvs = toks(g)
        for r in range(CHUNK_ROWS):
            tok = tvs[r // LANES][r % LANES]
            p = r * VOCAB_SIZE + tok
            q = (p >> 4) << 4
            buf[pl.ds(q, LANES)] = zeros16

    def dma(buf, sem, g):
        dst = out_hbm.at[pl.ds(flat_base + g * CHUNK_ELEMS, CHUNK_ELEMS)]
        return pltpu.make_async_copy(buf, dst, sem)

    for b in range(NBUF):
        marks(bufs[b], b, 1.0)
        dma(bufs[b], sems[b], b).start()

    def body(i, _):
        for b in range(NBUF):
            g = NBUF * i + b
            dma(bufs[b], sems[b], g - NBUF).wait()
            clear(bufs[b], g - NBUF)
            marks(bufs[b], g, 1.0)
            dma(bufs[b], sems[b], g).start()
        return 0

    lax.fori_loop(1, NUM_CHUNKS // NBUF, body, 0)

    for b in range(NBUF):
        dma(bufs[b], sems[b], NUM_CHUNKS - NBUF + b).wait()


_sc_kernel = pl.kernel(
    _sc_onehot,
    mesh=plsc.VectorSubcoreMesh(core_axis_name="c", subcore_axis_name="s"),
    out_type=jax.ShapeDtypeStruct((TOTAL_ROWS * VOCAB_SIZE,), jnp.float32),
    scratch_types=(
        [pltpu.VMEM((CHUNK_ELEMS,), jnp.float32) for _ in range(NBUF)]
        + [pltpu.VMEM((ROWS_PER_WORKER + LANES,), jnp.int32)]
        + [pltpu.SemaphoreType.DMA for _ in range(NBUF)]
    ),
)


def kernel(tokens, table):
    del table
    flat = _sc_kernel(tokens.reshape(-1))
    return flat.reshape(tokens.shape[0], tokens.shape[1], VOCAB_SIZE)


# speedup vs baseline: 2.0729x; 1.0031x over previous
import jax
import jax.numpy as jnp
from jax import lax
from jax.experimental import pallas as pl
from jax.experimental.pallas import tpu as pltpu
from jax.experimental.pallas import tpu_sc as plsc

VOCAB_SIZE = 1000
NUM_CORES = 2
NUM_SUBCORES = 16
NUM_WORKERS = NUM_CORES * NUM_SUBCORES
LANES = 16

TOTAL_ROWS = 4096 * 20
ROWS_PER_WORKER = TOTAL_ROWS // NUM_WORKERS
CHUNK_ROWS = 10
CHUNK_ELEMS = CHUNK_ROWS * VOCAB_SIZE
NUM_CHUNKS = ROWS_PER_WORKER // CHUNK_ROWS
NBUF = 8


def _sc_onehot(tok_hbm, out_hbm, *scratch):
    bufs = scratch[:NBUF]
    idx_v = scratch[NBUF]
    sems = scratch[NBUF + 1:]
    wid = lax.axis_index("s") * NUM_CORES + lax.axis_index("c")
    flat_base = wid * ROWS_PER_WORKER * VOCAB_SIZE
    tok_base = wid * ROWS_PER_WORKER
    pltpu.sync_copy(
        tok_hbm.at[pl.ds(tok_base, ROWS_PER_WORKER)],
        idx_v.at[pl.ds(0, ROWS_PER_WORKER)],
    )

    zeros16 = jnp.zeros((LANES,), jnp.float32)
    iota16 = lax.iota(jnp.int32, LANES)

    def zinit(i, _):
        for buf in bufs:
            buf[pl.ds(i * LANES, LANES)] = zeros16
        return 0

    lax.fori_loop(0, CHUNK_ELEMS // LANES, zinit, 0)

    def toks(g):
        return [
            idx_v[pl.ds(g * CHUNK_ROWS + k * LANES, LANES)]
            for k in range((CHUNK_ROWS + LANES - 1) // LANES)
        ]

    def marks(buf, g, value):
        tvs = toks(g)
        for r in range(CHUNK_ROWS):
            tok = tvs[r // LANES][r % LANES]
            p = r * VOCAB_SIZE + tok
            q = (p >> 4) << 4
            lane = p - q
            vec = buf[pl.ds(q, LANES)]
            vec = jnp.where(iota16 == lane, jnp.float32(value), vec)
            buf[pl.ds(q, LANES)] = vec

    def clear(buf, g):
        tvs = toks(g)
        for r in range(CHUNK_ROWS):
            tok = tvs[r // LANES][r % LANES]
            p = r * VOCAB_SIZE + tok
            q = (p >> 4) << 4
            buf[pl.ds(q, LANES)] = zeros16

    def dma(buf, sem, g):
        dst = out_hbm.at[pl.ds(flat_base + g * CHUNK_ELEMS, CHUNK_ELEMS)]
        return pltpu.make_async_copy(buf, dst, sem)

    for b in range(NBUF):
        marks(bufs[b], b, 1.0)
        dma(bufs[b], sems[b], b).start()

    def body(i, _):
        for b in range(NBUF):
            g = NBUF * i + b
            dma(bufs[b], sems[b], g - NBUF).wait()
            clear(bufs[b], g - NBUF)
            marks(bufs[b], g, 1.0)
            dma(bufs[b], sems[b], g).start()
        return 0

    lax.fori_loop(1, NUM_CHUNKS // NBUF, body, 0)

    for b in range(NBUF):
        dma(bufs[b], sems[b], NUM_CHUNKS - NBUF + b).wait()


_sc_kernel = pl.kernel(
    _sc_onehot,
    mesh=plsc.VectorSubcoreMesh(core_axis_name="c", subcore_axis_name="s"),
    out_type=jax.ShapeDtypeStruct((TOTAL_ROWS * VOCAB_SIZE,), jnp.float32),
    scratch_types=(
        [pltpu.VMEM((CHUNK_ELEMS,), jnp.float32) for _ in range(NBUF)]
        + [pltpu.VMEM((ROWS_PER_WORKER + LANES,), jnp.int32)]
        + [pltpu.SemaphoreType.DMA for _ in range(NBUF)]
    ),
)


def kernel(tokens, table):
    del table
    flat = _sc_kernel(tokens.reshape(-1))
    return flat.reshape(tokens.shape[0], tokens.shape[1], VOCAB_SIZE)


# speedup vs baseline: 2.0837x; 1.0052x over previous
import jax
import jax.numpy as jnp
from jax import lax
from jax.experimental import pallas as pl
from jax.experimental.pallas import tpu as pltpu
from jax.experimental.pallas import tpu_sc as plsc

VOCAB_SIZE = 1000
NUM_CORES = 2
NUM_SUBCORES = 16
NUM_WORKERS = NUM_CORES * NUM_SUBCORES
LANES = 16

TOTAL_ROWS = 4096 * 20
ROWS_PER_WORKER = TOTAL_ROWS // NUM_WORKERS
CHUNK_ROWS = 5
CHUNK_ELEMS = CHUNK_ROWS * VOCAB_SIZE
NUM_CHUNKS = ROWS_PER_WORKER // CHUNK_ROWS
NBUF = 8


def _sc_onehot(tok_hbm, out_hbm, *scratch):
    bufs = scratch[:NBUF]
    idx_v = scratch[NBUF]
    sems = scratch[NBUF + 1:]
    wid = lax.axis_index("s") * NUM_CORES + lax.axis_index("c")
    flat_base = wid * ROWS_PER_WORKER * VOCAB_SIZE
    tok_base = wid * ROWS_PER_WORKER
    pltpu.sync_copy(
        tok_hbm.at[pl.ds(tok_base, ROWS_PER_WORKER)],
        idx_v.at[pl.ds(0, ROWS_PER_WORKER)],
    )

    zeros16 = jnp.zeros((LANES,), jnp.float32)
    iota16 = lax.iota(jnp.int32, LANES)

    def zinit(i, _):
        for buf in bufs:
            buf[pl.ds(i * LANES, LANES)] = zeros16
        return 0

    lax.fori_loop(0, CHUNK_ELEMS // LANES, zinit, 0)

    def toks(g):
        return [
            idx_v[pl.ds(g * CHUNK_ROWS + k * LANES, LANES)]
            for k in range((CHUNK_ROWS + LANES - 1) // LANES)
        ]

    def marks(buf, g, value):
        tvs = toks(g)
        for r in range(CHUNK_ROWS):
            tok = tvs[r // LANES][r % LANES]
            p = r * VOCAB_SIZE + tok
            q = (p >> 4) << 4
            lane = p - q
            vec = buf[pl.ds(q, LANES)]
            vec = jnp.where(iota16 == lane, jnp.float32(value), vec)
            buf[pl.ds(q, LANES)] = vec

    def clear(buf, g):
        tvs = toks(g)
        for r in range(CHUNK_ROWS):
            tok = tvs[r // LANES][r % LANES]
            p = r * VOCAB_SIZE + tok
            q = (p >> 4) << 4
            buf[pl.ds(q, LANES)] = zeros16

    def dma(buf, sem, g):
        dst = out_hbm.at[pl.ds(flat_base + g * CHUNK_ELEMS, CHUNK_ELEMS)]
        return pltpu.make_async_copy(buf, dst, sem)

    for b in range(NBUF):
        marks(bufs[b], b, 1.0)
        dma(bufs[b], sems[b], b).start()

    def body(i, _):
        for b in range(NBUF):
            g = NBUF * i + b
            dma(bufs[b], sems[b], g - NBUF).wait()
            clear(bufs[b], g - NBUF)
            marks(bufs[b], g, 1.0)
            dma(bufs[b], sems[b], g).start()
        return 0

    lax.fori_loop(1, NUM_CHUNKS // NBUF, body, 0)

    for b in range(NBUF):
        dma(bufs[b], sems[b], NUM_CHUNKS - NBUF + b).wait()


_sc_kernel = pl.kernel(
    _sc_onehot,
    mesh=plsc.VectorSubcoreMesh(core_axis_name="c", subcore_axis_name="s"),
    out_type=jax.ShapeDtypeStruct((TOTAL_ROWS * VOCAB_SIZE,), jnp.float32),
    scratch_types=(
        [pltpu.VMEM((CHUNK_ELEMS,), jnp.float32) for _ in range(NBUF)]
        + [pltpu.VMEM((ROWS_PER_WORKER + LANES,), jnp.int32)]
        + [pltpu.SemaphoreType.DMA for _ in range(NBUF)]
    ),
)


def kernel(tokens, table):
    del table
    flat = _sc_kernel(tokens.reshape(-1))
    return flat.reshape(tokens.shape[0], tokens.shape[1], VOCAB_SIZE)
